# Initial kernel scaffold; baseline (speedup 1.0000x reference)
import jax
import jax.numpy as jnp
from jax.experimental import pallas as pl


def kernel(input, codebook):
    raise NotImplementedError("write your pallas kernel here")


# speedup vs baseline: 1.4469x; 1.4469x over previous
import functools

import jax
import jax.numpy as jnp
from jax import lax
from jax.experimental import pallas as pl
from jax.experimental.pallas import tpu as pltpu
from jax.experimental.pallas import tpu_sc as plsc

CODEBOOK_SIZE = 8192
EMBEDDING_DIM = 32

K_CHUNK = 2048

_NUM_CORES = 2
_NUM_SUBCORES = 16
_NUM_WORKERS = _NUM_CORES * _NUM_SUBCORES


def _argmin_body(x_ref, cb_ref, idx_ref):
    x = x_ref[0]
    hw = x.shape[1]
    x_sq = jnp.sum(x * x, axis=0)[None, :]
    xh = x.astype(jnp.bfloat16)

    def chunk(c, carry):
        run_min, run_idx = carry
        k0 = c * K_CHUNK
        cb = cb_ref[pl.ds(k0, K_CHUNK), :]
        c_sq = jnp.sum(cb * cb, axis=1)[:, None]
        scores = lax.dot_general(
            cb.astype(jnp.bfloat16), xh, (((1,), (0,)), ((), ())),
            preferred_element_type=jnp.float32)
        dist = (x_sq - 2.0 * scores) + c_sq
        cmin = jnp.min(dist, axis=0)
        rows = lax.broadcasted_iota(jnp.int32, dist.shape, 0)
        carg = jnp.min(jnp.where(dist == cmin[None, :], rows, CODEBOOK_SIZE),
                       axis=0) + k0
        better = cmin < run_min
        return (jnp.where(better, cmin, run_min),
                jnp.where(better, carg, run_idx))

    init = (jnp.full((hw,), jnp.inf, jnp.float32),
            jnp.zeros((hw,), jnp.int32))
    _, idx = lax.fori_loop(0, CODEBOOK_SIZE // K_CHUNK, chunk, init)
    idx_ref[0, 0, :] = idx


def _nearest_indices(x3, codebook):
    B, D, HW = x3.shape
    K = codebook.shape[0]
    return pl.pallas_call(
        _argmin_body,
        grid=(B,),
        in_specs=[
            pl.BlockSpec((1, D, HW), lambda b: (b, 0, 0)),
            pl.BlockSpec((K, D), lambda b: (0, 0)),
        ],
        out_specs=pl.BlockSpec((1, 1, HW), lambda b: (b, 0, 0)),
        out_shape=jax.ShapeDtypeStruct((B, 1, HW), jnp.int32),
    )(x3, codebook)


_GATHER_PAD = 128
_GATHER_CHUNK = 512


def _gather_body(idx_hbm, table_hbm, out_hbm, idx_v, rows_v, sem):
    wid = lax.axis_index("s") * _NUM_CORES + lax.axis_index("c")
    n = idx_v.shape[0]
    base = wid * n
    pltpu.sync_copy(idx_hbm.at[pl.ds(base, n)], idx_v)

    def step(c, carry):
        o = c * _GATHER_CHUNK
        pltpu.async_copy(
            table_hbm.at[idx_v.at[pl.ds(o, _GATHER_CHUNK)]], rows_v, sem
        ).wait()
        pltpu.sync_copy(rows_v, out_hbm.at[pl.ds(base + o, _GATHER_CHUNK)])
        return carry

    lax.fori_loop(0, n // _GATHER_CHUNK, step, 0)


def _gather_rows(idx, table_padded):
    T = idx.shape[0]
    per_w = T // _NUM_WORKERS
    mesh = plsc.VectorSubcoreMesh(core_axis_name="c", subcore_axis_name="s")
    return pl.kernel(
        _gather_body,
        out_type=jax.ShapeDtypeStruct((T, _GATHER_PAD), jnp.float32),
        mesh=mesh,
        scratch_types=[
            pltpu.VMEM((per_w,), jnp.int32),
            pltpu.VMEM((_GATHER_CHUNK, _GATHER_PAD), jnp.float32),
            pltpu.SemaphoreType.DMA,
        ],
    )(idx, table_padded)


def kernel(input, codebook):
    B, D = input.shape[0], input.shape[1]
    spatial = input.shape[2:]
    HW = 1
    for s in spatial:
        HW *= s
    x3 = input.reshape(B, D, HW)
    idx = _nearest_indices(x3, codebook).reshape(B * HW)
    table_padded = jnp.pad(codebook, ((0, 0), (0, _GATHER_PAD - D)))
    rows = _gather_rows(idx, table_padded)[:, :D]
    out = jnp.moveaxis(rows.reshape((B,) + spatial + (D,)), -1, 1)
    indices = idx.reshape((B,) + spatial).astype(jnp.int64)
    return out, indices
